# baseline (speedup 1.0000x reference)
import jax
import jax.numpy as jnp
from jax import lax
from jax.experimental import pallas as pl

D = 128
N = 100000
NT = 100
NT_PAD = 128
NB = 2000
NBLK = N // NB


def _tc_body(idx_ref, wt_ref, out_ref):
    ids = idx_ref[0, 0, :]
    tio = lax.broadcasted_iota(jnp.int16, (NT_PAD, NB), 0)
    oht = (tio == ids[None, :]).astype(jnp.bfloat16)
    out_ref[...] = lax.dot_general(
        oht,
        wt_ref[...],
        dimension_numbers=(((0,), (0,)), ((), ())),
        preferred_element_type=jnp.float32,
    )


def kernel(atom_types, W):
    ids3 = atom_types.astype(jnp.int16).reshape(NBLK, 1, NB)
    wt = jnp.zeros((NT_PAD, D), jnp.bfloat16).at[:NT, :].set(
        W.T.astype(jnp.bfloat16)
    )
    return pl.pallas_call(
        _tc_body,
        grid=(NBLK,),
        in_specs=[
            pl.BlockSpec((1, 1, NB), lambda g: (g, 0, 0)),
            pl.BlockSpec((NT_PAD, D), lambda g: (0, 0)),
        ],
        out_specs=pl.BlockSpec((NB, D), lambda g: (g, 0)),
        out_shape=jax.ShapeDtypeStruct((N, D), jnp.float32),
    )(ids3, wt)


# speedup vs baseline: 1.7894x; 1.7894x over previous
import jax
import jax.numpy as jnp
from jax import lax
from jax.experimental import pallas as pl
from jax.experimental.pallas import tpu as pltpu

D = 128
N = 100000
NT = 100
NT_PAD = 112
NB = 10000
NBLK = N // NB
PAR = 2


def _tc_body(idx_ref, wt_ref, out_ref, obuf, sem):
    g = pl.program_id(0)

    def compute(p):
        ids = idx_ref[pl.program_id(0), 0, :]
        tio = lax.broadcasted_iota(jnp.int16, (NB, NT_PAD), 1)
        onehot_bits = jnp.where(
            tio == ids[:, None], jnp.int16(0x3F80), jnp.int16(0)
        )
        oh = lax.bitcast_convert_type(onehot_bits, jnp.bfloat16)
        obuf[p] = lax.dot_general(
            oh,
            wt_ref[...],
            dimension_numbers=(((1,), (0,)), ((), ())),
            preferred_element_type=jnp.float32,
        )

    for p in range(PAR):

        @pl.when(g % PAR == p)
        def _():
            @pl.when(g >= PAR)
            def _wait_prev():
                pltpu.make_async_copy(
                    obuf.at[p], out_ref.at[pl.ds((g - PAR) * NB, NB)], sem.at[p]
                ).wait()

            compute(p)
            pltpu.make_async_copy(
                obuf.at[p], out_ref.at[pl.ds(g * NB, NB)], sem.at[p]
            ).start()

    @pl.when(g == NBLK - 1)
    def _drain():
        for p in range(PAR):
            blk = NBLK - PAR + p
            pltpu.make_async_copy(
                obuf.at[p], out_ref.at[pl.ds(blk * NB, NB)], sem.at[p]
            ).wait()


def kernel(atom_types, W):
    ids3 = atom_types.astype(jnp.int16).reshape(NBLK, 1, NB)
    wt = jnp.zeros((NT_PAD, D), jnp.bfloat16).at[:NT, :].set(
        W.T.astype(jnp.bfloat16)
    )
    return pl.pallas_call(
        _tc_body,
        grid=(NBLK,),
        in_specs=[
            pl.BlockSpec((NBLK, 1, NB), lambda g: (0, 0, 0)),
            pl.BlockSpec((NT_PAD, D), lambda g: (0, 0)),
        ],
        out_specs=pl.BlockSpec(memory_space=pltpu.MemorySpace.HBM),
        out_shape=jax.ShapeDtypeStruct((N, D), jnp.float32),
        scratch_shapes=[
            pltpu.VMEM((PAR, NB, D), jnp.float32),
            pltpu.SemaphoreType.DMA((PAR,)),
        ],
    )(ids3, wt)
